# baseline (speedup 1.0000x reference)
import math

import jax
import jax.numpy as jnp
import numpy as np
from jax.experimental import pallas as pl
from jax.experimental.pallas import tpu as pltpu

BS, TLEN, NNODE, DIM = 128, 32, 32, 9
K = 3
LSTMH, LSTMO = 32, 16
CONV_OUT = DIM - 2 * (K - 1)
C1OUT = DIM - (K - 1)
HID = 32
D2 = 2 * HID
WIN = (4, 8)
STR = (2, 4)
DECAY = 0.7
EPS = 1e-5
NW1 = (TLEN - WIN[0]) // STR[0] + 1
NW2 = (TLEN - WIN[1]) // STR[1] + 1
NW = NW1 + NW2
ROWS = TLEN * NNODE
BB = 8

_INV = 1.0 / math.sqrt(1.0 + EPS)


def _pos_encoding_np(tlen, d):
    pos = np.arange(tlen, dtype=np.float32)[:, None]
    div = np.exp(np.arange(0, d, 2, dtype=np.float32) * (-math.log(10000.0) / d))
    pe = np.zeros((tlen, d), dtype=np.float32)
    pe[:, 0::2] = np.sin(pos * div)
    pe[:, 1::2] = np.cos(pos * div)
    return pe


def _decay_mask_np(w, nnode):
    ti = np.repeat(np.arange(w), nnode).astype(np.float32)
    return (DECAY ** np.abs(ti[:, None] - ti[None, :])).astype(np.float32)


_T1 = np.zeros((K, DIM, C1OUT), dtype=np.float32)
for _k in range(K):
    for _t in range(C1OUT):
        _T1[_k, _t + _k, _t] = 1.0
_T2 = np.zeros((K, C1OUT, CONV_OUT), dtype=np.float32)
for _k in range(K):
    for _t in range(CONV_OUT):
        _T2[_k, _t + _k, _t] = 1.0

_PE_REP = np.repeat(_pos_encoding_np(TLEN, D2), NNODE, axis=0)
_MASK1 = _decay_mask_np(WIN[0], NNODE)
_MASK2 = _decay_mask_np(WIN[1], NNODE)
_N1_ = WIN[0] * NNODE
_N2_ = WIN[1] * NNODE
_DNEG1 = (np.eye(_N1_, dtype=np.float32) * -1e30)
_DNEG2 = (np.eye(_N2_, dtype=np.float32) * -1e30)
_EYE1 = np.eye(_N1_, dtype=np.float32)
_EYE2 = np.eye(_N2_, dtype=np.float32)

_N1 = WIN[0] * NNODE
_N2 = WIN[1] * NNODE


def _leaky(x):
    return jnp.maximum(x, 0.01 * x)


def _bdot(a, b):
    return jnp.dot(a.astype(jnp.bfloat16), b.astype(jnp.bfloat16),
                   preferred_element_type=jnp.float32)


def _mpnn_kernel(x_ref, a_ref, s1_ref, b1_ref, b_ref, s2_ref, b2_ref,
                 m_ref, s3_ref, b3_ref,
                 gc1_ref, gc1b_ref, gc2_ref, gc2b_ref,
                 sx1_ref, bx1_ref, sx2_ref, bx2_ref,
                 th1_ref, so1_ref, bo1_ref, th2_ref, so2_ref, bo2_ref,
                 mask1_ref, mask2_ref, dneg1_ref, eye1_ref,
                 dneg2_ref, eye2_ref,
                 out_ref):
    f32 = jnp.float32
    x = x_ref[...].reshape(BB * ROWS, DIM)

    r1 = jnp.maximum(_bdot(x, a_ref[...]) * s1_ref[...] + b1_ref[...], 0.0)
    r2 = jnp.maximum(_bdot(r1, b_ref[...]) * s2_ref[...] + b2_ref[...], 0.0)
    e = _bdot(r2, m_ref[...]) * s3_ref[...] + b3_ref[...]

    nf1 = (_bdot(e, gc1_ref[...]) + gc1b_ref[...]).astype(jnp.bfloat16)
    nf2 = (_bdot(e, gc2_ref[...]) + gc2b_ref[...]).astype(jnp.bfloat16)
    xb1 = (e * sx1_ref[...] + bx1_ref[...]).astype(jnp.bfloat16)
    xb2 = (e * sx2_ref[...] + bx2_ref[...]).astype(jnp.bfloat16)

    def window_h0(g, xb, gstart, base, n, mask_ref, dneg_ref, eye_ref):
        start = base + gstart
        adj = g[gstart:gstart + n, gstart:gstart + n]
        ii = jax.lax.broadcasted_iota(jnp.int32, (n, n), 0)
        jj = jax.lax.broadcasted_iota(jnp.int32, (n, n), 1)
        diag = ii == jj
        adj = jnp.where(diag, -1e30, _leaky(adj))
        adj = adj - jnp.max(adj, axis=-1, keepdims=True)
        ex = jnp.exp(adj)
        sm = ex / jnp.sum(ex, axis=-1, keepdims=True)
        adjh = (sm * mask_ref[...]
                + jnp.where(diag, 1.0, 0.0)).astype(jnp.bfloat16)
        return jnp.dot(adjh, xb[start:start + n, :],
                       preferred_element_type=f32)

    h0a_list = []
    h0b_list = []
    for b in range(BB):
        base = b * ROWS
        g1 = jnp.dot(nf1[base:base + ROWS, :], nf1[base:base + ROWS, :].T,
                     preferred_element_type=f32)
        g2 = jnp.dot(nf2[base:base + ROWS, :], nf2[base:base + ROWS, :].T,
                     preferred_element_type=f32)
        for wi in range(NW1):
            h0a_list.append(window_h0(g1, xb1, wi * STR[0] * NNODE, base,
                                      _N1, mask1_ref, dneg1_ref, eye1_ref))
        for wj in range(NW2):
            h0b_list.append(window_h0(g2, xb2, wj * STR[1] * NNODE, base,
                                      _N2, mask2_ref, dneg2_ref, eye2_ref))

    h0a = jnp.concatenate(h0a_list, axis=0)
    h0b = jnp.concatenate(h0b_list, axis=0)

    ha = _leaky(_bdot(h0a, th1_ref[...]) * so1_ref[...] + bo1_ref[...])
    hb = _leaky(_bdot(h0b, th2_ref[...]) * so2_ref[...] + bo2_ref[...])

    hma = ha.reshape(BB * NW1, WIN[0], NNODE, HID).mean(axis=1)
    hmb = hb.reshape(BB * NW2, WIN[1], NNODE, HID).mean(axis=1)
    out_ref[:, 0:NW1 * NNODE, :] = hma.reshape(BB, NW1 * NNODE, HID)
    out_ref[:, NW1 * NNODE:NW * NNODE, :] = hmb.reshape(BB, NW2 * NNODE, HID)


def _head_kernel(f_ref, w1_ref, b1_ref, w2_ref, b2_ref, w3_ref, b3_ref,
                 w4_ref, b4_ref, out_ref):
    z = jnp.maximum(_bdot(f_ref[...], w1_ref[...]) + b1_ref[...], 0.0)
    z = jnp.maximum(_bdot(z, w2_ref[...]) + b2_ref[...], 0.0)
    z = jnp.maximum(_bdot(z, w3_ref[...]) + b3_ref[...], 0.0)
    out_ref[...] = _bdot(z, w4_ref[...]) + b4_ref[...]


def kernel(X, params):
    p = params
    f32 = jnp.float32

    w1 = p['conv1_w'][:, 0, :]
    A = jnp.einsum('kjt,ck->jct', jnp.asarray(_T1), w1).reshape(DIM, LSTMH * C1OUT)
    s1c = jnp.repeat(p['bn1_g'] * _INV, C1OUT)
    b1c = jnp.repeat(p['conv1_b'] * p['bn1_g'] * _INV + p['bn1_b'], C1OUT)

    B = jnp.einsum('kjt,ock->cjot', jnp.asarray(_T2),
                   p['conv2_w']).reshape(LSTMH * C1OUT, LSTMO * CONV_OUT)
    s2c = jnp.repeat(p['bn2_g'] * _INV, CONV_OUT)
    b2c = jnp.repeat(p['conv2_b'] * p['bn2_g'] * _INV + p['bn2_b'], CONV_OUT)

    s3 = p['map2_bn_g'] * _INV
    b3 = p['map2_b'] * s3 + p['map2_bn_b']
    b3pe = jnp.tile(jnp.asarray(_PE_REP) + b3[None, :], (BB, 1))

    sbn1 = p['m1_bn_g'] * _INV
    sbn2 = p['m2_bn_g'] * _INV
    so1 = p['m1_obn_g'] * _INV
    so2 = p['m2_obn_g'] * _INV
    bo1 = p['m1_th_b'] * so1 + p['m1_obn_b']
    bo2 = p['m2_th_b'] * so2 + p['m2_obn_b']

    def v(x):
        return x.reshape(1, -1).astype(f32)

    inputs = [
        X,
        A, v(s1c), v(b1c), B, v(s2c), v(b2c),
        p['map2_w'].T, v(s3), b3pe,
        p['m1_gc_w'].T, v(p['m1_gc_b']), p['m2_gc_w'].T, v(p['m2_gc_b']),
        v(sbn1), v(p['m1_bn_b']), v(sbn2), v(p['m2_bn_b']),
        p['m1_th_w'].T, v(so1), v(bo1), p['m2_th_w'].T, v(so2), v(bo2),
        jnp.asarray(_MASK1), jnp.asarray(_MASK2),
        jnp.asarray(_DNEG1), jnp.asarray(_EYE1),
        jnp.asarray(_DNEG2), jnp.asarray(_EYE2),
    ]

    def whole(a):
        nd = a.ndim
        return pl.BlockSpec(a.shape, lambda b, _n=nd: (0,) * _n)

    in_specs = [pl.BlockSpec((BB, TLEN, NNODE, DIM), lambda b: (b, 0, 0, 0))]
    in_specs += [whole(a) for a in inputs[1:]]

    H = pl.pallas_call(
        _mpnn_kernel,
        grid=(BS // BB,),
        in_specs=in_specs,
        out_specs=pl.BlockSpec((BB, NW * NNODE, HID), lambda b: (b, 0, 0)),
        out_shape=jax.ShapeDtypeStruct((BS, NW * NNODE, HID), f32),
        compiler_params=pltpu.CompilerParams(
            dimension_semantics=("parallel",)),
    )(*inputs)

    F = H.reshape(BS, NW * NNODE * HID)

    head_inputs = [
        F,
        p['fc1_w'].T, v(p['fc1_b']),
        p['fc2_w'].T, v(p['fc2_b']),
        p['fc3_w'].T, v(p['fc3_b']),
        p['fc4_w'].T, v(p['fc4_b']),
    ]
    out = pl.pallas_call(
        _head_kernel,
        out_shape=jax.ShapeDtypeStruct((BS, 1), f32),
    )(*head_inputs)
    return out


# speedup vs baseline: 1.0124x; 1.0124x over previous
import math

import jax
import jax.numpy as jnp
import numpy as np
from jax.experimental import pallas as pl
from jax.experimental.pallas import tpu as pltpu

BS, TLEN, NNODE, DIM = 128, 32, 32, 9
K = 3
LSTMH, LSTMO = 32, 16
CONV_OUT = DIM - 2 * (K - 1)
C1OUT = DIM - (K - 1)
HID = 32
D2 = 2 * HID
WIN = (4, 8)
STR = (2, 4)
DECAY = 0.7
EPS = 1e-5
NW1 = (TLEN - WIN[0]) // STR[0] + 1
NW2 = (TLEN - WIN[1]) // STR[1] + 1
NW = NW1 + NW2
ROWS = TLEN * NNODE
BB = 4

_INV = 1.0 / math.sqrt(1.0 + EPS)


def _pos_encoding_np(tlen, d):
    pos = np.arange(tlen, dtype=np.float32)[:, None]
    div = np.exp(np.arange(0, d, 2, dtype=np.float32) * (-math.log(10000.0) / d))
    pe = np.zeros((tlen, d), dtype=np.float32)
    pe[:, 0::2] = np.sin(pos * div)
    pe[:, 1::2] = np.cos(pos * div)
    return pe


def _decay_mask_np(w, nnode):
    ti = np.repeat(np.arange(w), nnode).astype(np.float32)
    return (DECAY ** np.abs(ti[:, None] - ti[None, :])).astype(np.float32)


_T1 = np.zeros((K, DIM, C1OUT), dtype=np.float32)
for _k in range(K):
    for _t in range(C1OUT):
        _T1[_k, _t + _k, _t] = 1.0
_T2 = np.zeros((K, C1OUT, CONV_OUT), dtype=np.float32)
for _k in range(K):
    for _t in range(CONV_OUT):
        _T2[_k, _t + _k, _t] = 1.0

_PE_REP = np.repeat(_pos_encoding_np(TLEN, D2), NNODE, axis=0)
_MASK1 = _decay_mask_np(WIN[0], NNODE)
_MASK2 = _decay_mask_np(WIN[1], NNODE)
_N1_ = WIN[0] * NNODE
_N2_ = WIN[1] * NNODE
_DNEG1 = (np.eye(_N1_, dtype=np.float32) * -1e30)
_DNEG2 = (np.eye(_N2_, dtype=np.float32) * -1e30)
_EYE1 = np.eye(_N1_, dtype=np.float32)
_EYE2 = np.eye(_N2_, dtype=np.float32)

_N1 = WIN[0] * NNODE
_N2 = WIN[1] * NNODE


def _leaky(x):
    return jnp.maximum(x, 0.01 * x)


def _bdot(a, b):
    return jnp.dot(a.astype(jnp.bfloat16), b.astype(jnp.bfloat16),
                   preferred_element_type=jnp.float32)


def _mpnn_kernel(x_ref, a_ref, s1_ref, b1_ref, b_ref, s2_ref, b2_ref,
                 m_ref, s3_ref, b3_ref,
                 gc1_ref, gc1b_ref, gc2_ref, gc2b_ref,
                 sx1_ref, bx1_ref, sx2_ref, bx2_ref,
                 th1_ref, so1_ref, bo1_ref, th2_ref, so2_ref, bo2_ref,
                 mask1_ref, mask2_ref, dneg1_ref, eye1_ref,
                 dneg2_ref, eye2_ref,
                 out_ref):
    f32 = jnp.float32
    x = x_ref[...].reshape(BB * ROWS, DIM)

    r1 = jnp.maximum(_bdot(x, a_ref[...]) * s1_ref[...] + b1_ref[...], 0.0)
    r2 = jnp.maximum(_bdot(r1, b_ref[...]) * s2_ref[...] + b2_ref[...], 0.0)
    e = _bdot(r2, m_ref[...]) * s3_ref[...] + b3_ref[...]

    nf1 = (_bdot(e, gc1_ref[...]) + gc1b_ref[...]).astype(jnp.bfloat16)
    nf2 = (_bdot(e, gc2_ref[...]) + gc2b_ref[...]).astype(jnp.bfloat16)
    xb1 = (e * sx1_ref[...] + bx1_ref[...]).astype(jnp.bfloat16)
    xb2 = (e * sx2_ref[...] + bx2_ref[...]).astype(jnp.bfloat16)

    def window_h0(g, xb, gstart, base, n, mask_ref, dneg_ref, eye_ref):
        start = base + gstart
        adj = g[gstart:gstart + n, gstart:gstart + n]
        ii = jax.lax.broadcasted_iota(jnp.int32, (n, n), 0)
        jj = jax.lax.broadcasted_iota(jnp.int32, (n, n), 1)
        diag = ii == jj
        adj = jnp.where(diag, -1e30, _leaky(adj))
        adj = adj - jnp.max(adj, axis=-1, keepdims=True)
        ex = jnp.exp(adj)
        sm = ex / jnp.sum(ex, axis=-1, keepdims=True)
        adjh = (sm * mask_ref[...]
                + jnp.where(diag, 1.0, 0.0)).astype(jnp.bfloat16)
        return jnp.dot(adjh, xb[start:start + n, :],
                       preferred_element_type=f32)

    h0a_list = []
    h0b_list = []
    for b in range(BB):
        base = b * ROWS
        g1 = jnp.dot(nf1[base:base + ROWS, :], nf1[base:base + ROWS, :].T,
                     preferred_element_type=f32)
        g2 = jnp.dot(nf2[base:base + ROWS, :], nf2[base:base + ROWS, :].T,
                     preferred_element_type=f32)
        for wi in range(NW1):
            h0a_list.append(window_h0(g1, xb1, wi * STR[0] * NNODE, base,
                                      _N1, mask1_ref, dneg1_ref, eye1_ref))
        for wj in range(NW2):
            h0b_list.append(window_h0(g2, xb2, wj * STR[1] * NNODE, base,
                                      _N2, mask2_ref, dneg2_ref, eye2_ref))

    h0a = jnp.concatenate(h0a_list, axis=0)
    h0b = jnp.concatenate(h0b_list, axis=0)

    ha = _leaky(_bdot(h0a, th1_ref[...]) * so1_ref[...] + bo1_ref[...])
    hb = _leaky(_bdot(h0b, th2_ref[...]) * so2_ref[...] + bo2_ref[...])

    hma = ha.reshape(BB * NW1, WIN[0], NNODE, HID).mean(axis=1)
    hmb = hb.reshape(BB * NW2, WIN[1], NNODE, HID).mean(axis=1)
    out_ref[:, 0:NW1 * NNODE, :] = hma.reshape(BB, NW1 * NNODE, HID)
    out_ref[:, NW1 * NNODE:NW * NNODE, :] = hmb.reshape(BB, NW2 * NNODE, HID)


def _head_kernel(f_ref, w1_ref, b1_ref, w2_ref, b2_ref, w3_ref, b3_ref,
                 w4_ref, b4_ref, out_ref):
    z = jnp.maximum(_bdot(f_ref[...], w1_ref[...]) + b1_ref[...], 0.0)
    z = jnp.maximum(_bdot(z, w2_ref[...]) + b2_ref[...], 0.0)
    z = jnp.maximum(_bdot(z, w3_ref[...]) + b3_ref[...], 0.0)
    out_ref[...] = _bdot(z, w4_ref[...]) + b4_ref[...]


def kernel(X, params):
    p = params
    f32 = jnp.float32

    w1 = p['conv1_w'][:, 0, :]
    A = jnp.einsum('kjt,ck->jct', jnp.asarray(_T1), w1).reshape(DIM, LSTMH * C1OUT)
    s1c = jnp.repeat(p['bn1_g'] * _INV, C1OUT)
    b1c = jnp.repeat(p['conv1_b'] * p['bn1_g'] * _INV + p['bn1_b'], C1OUT)

    B = jnp.einsum('kjt,ock->cjot', jnp.asarray(_T2),
                   p['conv2_w']).reshape(LSTMH * C1OUT, LSTMO * CONV_OUT)
    s2c = jnp.repeat(p['bn2_g'] * _INV, CONV_OUT)
    b2c = jnp.repeat(p['conv2_b'] * p['bn2_g'] * _INV + p['bn2_b'], CONV_OUT)

    s3 = p['map2_bn_g'] * _INV
    b3 = p['map2_b'] * s3 + p['map2_bn_b']
    b3pe = jnp.tile(jnp.asarray(_PE_REP) + b3[None, :], (BB, 1))

    sbn1 = p['m1_bn_g'] * _INV
    sbn2 = p['m2_bn_g'] * _INV
    so1 = p['m1_obn_g'] * _INV
    so2 = p['m2_obn_g'] * _INV
    bo1 = p['m1_th_b'] * so1 + p['m1_obn_b']
    bo2 = p['m2_th_b'] * so2 + p['m2_obn_b']

    def v(x):
        return x.reshape(1, -1).astype(f32)

    inputs = [
        X,
        A, v(s1c), v(b1c), B, v(s2c), v(b2c),
        p['map2_w'].T, v(s3), b3pe,
        p['m1_gc_w'].T, v(p['m1_gc_b']), p['m2_gc_w'].T, v(p['m2_gc_b']),
        v(sbn1), v(p['m1_bn_b']), v(sbn2), v(p['m2_bn_b']),
        p['m1_th_w'].T, v(so1), v(bo1), p['m2_th_w'].T, v(so2), v(bo2),
        jnp.asarray(_MASK1), jnp.asarray(_MASK2),
        jnp.asarray(_DNEG1), jnp.asarray(_EYE1),
        jnp.asarray(_DNEG2), jnp.asarray(_EYE2),
    ]

    def whole(a):
        nd = a.ndim
        return pl.BlockSpec(a.shape, lambda b, _n=nd: (0,) * _n)

    in_specs = [pl.BlockSpec((BB, TLEN, NNODE, DIM), lambda b: (b, 0, 0, 0))]
    in_specs += [whole(a) for a in inputs[1:]]

    H = pl.pallas_call(
        _mpnn_kernel,
        grid=(BS // BB,),
        in_specs=in_specs,
        out_specs=pl.BlockSpec((BB, NW * NNODE, HID), lambda b: (b, 0, 0)),
        out_shape=jax.ShapeDtypeStruct((BS, NW * NNODE, HID), f32),
        compiler_params=pltpu.CompilerParams(
            dimension_semantics=("parallel",)),
    )(*inputs)

    F = H.reshape(BS, NW * NNODE * HID)

    head_inputs = [
        F,
        p['fc1_w'].T, v(p['fc1_b']),
        p['fc2_w'].T, v(p['fc2_b']),
        p['fc3_w'].T, v(p['fc3_b']),
        p['fc4_w'].T, v(p['fc4_b']),
    ]
    out = pl.pallas_call(
        _head_kernel,
        out_shape=jax.ShapeDtypeStruct((BS, 1), f32),
    )(*head_inputs)
    return out
